# baseline (speedup 1.0000x reference)
import functools

import jax
import jax.numpy as jnp
from jax import lax
from jax.experimental import pallas as pl
from jax.experimental.pallas import tpu as pltpu
from jax.experimental.pallas import tpu_sc as plsc

VOCAB = 1000000
B = 4096
L = 200
D = 16
NC = 2
NS = 16
NW = NC * NS
RW = B // NW
TPW = RW * L
GSZ = 128
G = TPW // GSZ
NBUF = 4
ACCR = RW + 8

_mesh = plsc.VectorSubcoreMesh(core_axis_name="c", subcore_axis_name="s")


@functools.partial(
    pl.kernel,
    out_type=jax.ShapeDtypeStruct((B, D), jnp.float32),
    mesh=_mesh,
    scratch_types=[
        pltpu.VMEM((G, GSZ), jnp.int32),
        pltpu.VMEM((G, GSZ), jnp.int32),
        pltpu.VMEM((ACCR, D), jnp.float32),
        pltpu.VMEM((NBUF, GSZ, D), jnp.float32),
        pltpu.VMEM_SHARED((NS * ACCR, D), jnp.float32),
        [pltpu.SemaphoreType.DMA] * NBUF,
    ],
    compiler_params=pltpu.CompilerParams(use_tc_tiling_on_sc=False),
)
def _sc_pool(ids_hbm, mask_hbm, table_hbm, out_hbm,
             ids_v, dst_v, zero_v, buf_v, acc_sh, sems):
    c = lax.axis_index("c")
    s = lax.axis_index("s")
    wid = c * NS + s
    base = s * ACCR
    junk = base + RW

    pltpu.sync_copy(ids_hbm.at[pl.ds(wid * G, G)], ids_v)
    pltpu.sync_copy(mask_hbm.at[pl.ds(wid * G, G)], dst_v)

    def zrow(i, _):
        zero_v[i, :] = jnp.zeros((D,), jnp.float32)
        return ()
    lax.fori_loop(0, ACCR, zrow, ())
    pltpu.sync_copy(zero_v, acc_sh.at[pl.ds(base, ACCR)])

    lanes = lax.iota(jnp.int32, 16)
    zeros16 = jnp.zeros((16,), jnp.int32)
    lvec = jnp.full((16,), L, jnp.int32)
    basev = jnp.full((16,), base, jnp.int32)
    junkv = jnp.full((16,), junk, jnp.int32)

    def dsti(i, _):
        for j in range(GSZ // 16):
            m = dst_v[i, pl.ds(j * 16, 16)]
            t = lanes + jnp.full((16,), i * GSZ + j * 16, jnp.int32)
            row = basev + lax.div(t, lvec)
            dst_v[i, pl.ds(j * 16, 16)] = jnp.where(m > zeros16, row, junkv)
        return ()
    lax.fori_loop(0, G, dsti, ())

    def gather(g, k):
        pltpu.make_async_copy(
            table_hbm.at[ids_v.at[g]], buf_v.at[k], sems[k]).start()

    def wait_scatter(g, k):
        pltpu.make_async_copy(
            table_hbm.at[ids_v.at[g]], buf_v.at[k], sems[k]).wait()
        pltpu.sync_copy(buf_v.at[k], acc_sh.at[dst_v.at[g]], add=True)

    for k in range(NBUF):
        gather(k, k)

    def ring(it, _):
        g0 = it * NBUF
        for k in range(NBUF):
            wait_scatter(g0 + k, k)
            gather(g0 + NBUF + k, k)
        return ()
    lax.fori_loop(0, G // NBUF - 1, ring, ())

    for k in range(NBUF):
        wait_scatter(G - NBUF + k, k)

    pltpu.sync_copy(acc_sh.at[pl.ds(base, RW)],
                    out_hbm.at[pl.ds(wid * RW, RW)])


TBLK = 512
TGRID = -(-VOCAB // TBLK)
VPAD = TGRID * TBLK


def _tc_transpose_body(t_ref, o_ref):
    o_ref[...] = t_ref[...].T


_tc_linearize = pl.pallas_call(
    _tc_transpose_body,
    grid=(TGRID,),
    in_specs=[pl.BlockSpec((D, TBLK), lambda k: (0, k))],
    out_specs=pl.BlockSpec((TBLK, D), lambda k: (k, 0)),
    out_shape=jax.ShapeDtypeStruct((VPAD, D), jnp.float32),
)


def kernel(input_ids, attention_mask, token_type_ids, embed_table):
    del token_type_ids
    ids = input_ids.astype(jnp.int32).reshape(B * L // GSZ, GSZ)
    mask = attention_mask.astype(jnp.int32).reshape(B * L // GSZ, GSZ)
    table_lin = lax.optimization_barrier(
        _tc_linearize(embed_table.T).reshape(VPAD * D)).reshape(VPAD, D)
    return _sc_pool(ids, mask, table_lin)


# speedup vs baseline: 2.3434x; 2.3434x over previous
import functools

import jax
import jax.numpy as jnp
from jax import lax
from jax.experimental import pallas as pl
from jax.experimental.pallas import tpu as pltpu
from jax.experimental.pallas import tpu_sc as plsc

VOCAB = 1000000
B = 4096
L = 200
D = 16
NC = 2
NS = 16
NW = NC * NS
RW = B // NW
TPW = RW * L
GSZ = 128
G = TPW // GSZ
NBUF = 4
ACCR = RW + 8

_mesh = plsc.VectorSubcoreMesh(core_axis_name="c", subcore_axis_name="s")


@functools.partial(
    pl.kernel,
    out_type=jax.ShapeDtypeStruct((B, D), jnp.float32),
    mesh=_mesh,
    scratch_types=[
        pltpu.VMEM((G, GSZ), jnp.int32),
        pltpu.VMEM((G, GSZ), jnp.int32),
        pltpu.VMEM((ACCR, D), jnp.float32),
        pltpu.VMEM((NBUF, GSZ, D), jnp.float32),
        pltpu.VMEM_SHARED((NS * ACCR, D), jnp.float32),
        [pltpu.SemaphoreType.DMA] * NBUF,
    ],
    compiler_params=pltpu.CompilerParams(use_tc_tiling_on_sc=False),
)
def _sc_pool(ids_hbm, mask_hbm, table_hbm, out_hbm,
             ids_v, dst_v, zero_v, buf_v, acc_sh, sems):
    c = lax.axis_index("c")
    s = lax.axis_index("s")
    wid = c * NS + s
    base = s * ACCR
    junk = base + RW

    pltpu.sync_copy(ids_hbm.at[pl.ds(wid * G, G)], ids_v)
    pltpu.sync_copy(mask_hbm.at[pl.ds(wid * G, G)], dst_v)

    def zrow(i, _):
        zero_v[i, :] = jnp.zeros((D,), jnp.float32)
        return ()
    lax.fori_loop(0, ACCR, zrow, ())
    pltpu.sync_copy(zero_v, acc_sh.at[pl.ds(base, ACCR)])

    lanes = lax.iota(jnp.int32, 16)
    zeros16 = jnp.zeros((16,), jnp.int32)
    lvec = jnp.full((16,), L, jnp.int32)
    basev = jnp.full((16,), base, jnp.int32)
    junkv = jnp.full((16,), junk, jnp.int32)
    hi_m = jnp.full((16,), -1024, jnp.int32)
    lo_m = jnp.full((16,), 127, jnp.int32)
    g_m = jnp.full((16,), 7, jnp.int32)
    c3 = jnp.full((16,), 3, jnp.int32)
    c7 = jnp.full((16,), 7, jnp.int32)

    def dsti(i, _):
        for j in range(GSZ // 16):
            m = dst_v[i, pl.ds(j * 16, 16)]
            t = lanes + jnp.full((16,), i * GSZ + j * 16, jnp.int32)
            row = basev + lax.div(t, lvec)
            dst_v[i, pl.ds(j * 16, 16)] = jnp.where(m > zeros16, row, junkv)
            v = ids_v[i, pl.ds(j * 16, 16)]
            idx2 = ((v & hi_m)
                    | lax.shift_left(v & lo_m, c3)
                    | (lax.shift_right_logical(v, c7) & g_m))
            ids_v[i, pl.ds(j * 16, 16)] = idx2
        return ()
    lax.fori_loop(0, G, dsti, ())

    def gather(g, k):
        pltpu.make_async_copy(
            table_hbm.at[ids_v.at[g]], buf_v.at[k], sems[k]).start()

    def wait_scatter(g, k):
        pltpu.make_async_copy(
            table_hbm.at[ids_v.at[g]], buf_v.at[k], sems[k]).wait()
        pltpu.sync_copy(buf_v.at[k], acc_sh.at[dst_v.at[g]], add=True)

    for k in range(NBUF):
        gather(k, k)

    def ring(it, _):
        g0 = it * NBUF
        for k in range(NBUF):
            wait_scatter(g0 + k, k)
            gather(g0 + NBUF + k, k)
        return ()
    lax.fori_loop(0, G // NBUF - 1, ring, ())

    for k in range(NBUF):
        wait_scatter(G - NBUF + k, k)

    pltpu.sync_copy(acc_sh.at[pl.ds(base, RW)],
                    out_hbm.at[pl.ds(wid * RW, RW)])


TBLK = 1024
TGRID = -(-VOCAB // TBLK)
VPAD = TGRID * TBLK


def _tc_transpose_body(t_ref, o_ref):
    x = t_ref[...]
    xs = jnp.concatenate(
        [x[:, 128 * g:128 * (g + 1)] for g in range(8)], axis=0)
    o_ref[...] = xs.T


_tc_linearize = pl.pallas_call(
    _tc_transpose_body,
    grid=(TGRID,),
    in_specs=[pl.BlockSpec((D, TBLK), lambda k: (0, k))],
    out_specs=pl.BlockSpec((TBLK * D // 128, 128), lambda k: (k, 0)),
    out_shape=jax.ShapeDtypeStruct((VPAD * D // 128, 128), jnp.float32),
)


def kernel(input_ids, attention_mask, token_type_ids, embed_table):
    del token_type_ids
    ids = input_ids.astype(jnp.int32).reshape(B * L // GSZ, GSZ)
    mask = attention_mask.astype(jnp.int32).reshape(B * L // GSZ, GSZ)
    table_lin = lax.optimization_barrier(
        _tc_linearize(embed_table.T).reshape(VPAD * D)).reshape(VPAD, D)
    return _sc_pool(ids, mask, table_lin)


# speedup vs baseline: 6.6673x; 2.8451x over previous
import functools

import jax
import jax.numpy as jnp
from jax import lax
from jax.experimental import pallas as pl
from jax.experimental.pallas import tpu as pltpu
from jax.experimental.pallas import tpu_sc as plsc

VOCAB = 1000000
B = 4096
L = 200
D = 16
NC = 2
NS = 16
NW = NC * NS
RW = B // NW
TPW = RW * L
GSZ = 128
G = TPW // GSZ
NBUF = 4
ACCR = RW + 8

_mesh = plsc.VectorSubcoreMesh(core_axis_name="c", subcore_axis_name="s")


@functools.partial(
    pl.kernel,
    out_type=jax.ShapeDtypeStruct((B, D), jnp.float32),
    mesh=_mesh,
    scratch_types=[
        pltpu.VMEM((G, GSZ), jnp.int32),
        pltpu.VMEM((G, GSZ), jnp.int32),
        pltpu.VMEM((ACCR, D), jnp.float32),
        pltpu.VMEM((NBUF, GSZ, D), jnp.float32),
        pltpu.VMEM_SHARED((NS * ACCR, D), jnp.float32),
        [pltpu.SemaphoreType.DMA] * NBUF,
    ],
    compiler_params=pltpu.CompilerParams(use_tc_tiling_on_sc=False),
)
def _sc_pool(ids_hbm, mask_hbm, table_hbm, out_hbm,
             ids_v, dst_v, zero_v, buf_v, acc_sh, sems):
    c = lax.axis_index("c")
    s = lax.axis_index("s")
    wid = c * NS + s
    base = s * ACCR
    junk = base + RW

    pltpu.sync_copy(ids_hbm.at[pl.ds(wid * G, G)], ids_v)
    pltpu.sync_copy(mask_hbm.at[pl.ds(wid * G, G)], dst_v)

    def zrow(i, _):
        zero_v[i, :] = jnp.zeros((D,), jnp.float32)
        return ()
    lax.fori_loop(0, ACCR, zrow, ())
    pltpu.sync_copy(zero_v, acc_sh.at[pl.ds(base, ACCR)])

    lanes = lax.iota(jnp.int32, 16)
    zeros16 = jnp.zeros((16,), jnp.int32)
    lvec = jnp.full((16,), L, jnp.int32)
    basev = jnp.full((16,), base, jnp.int32)
    junkv = jnp.full((16,), junk, jnp.int32)
    hi_m = jnp.full((16,), -1024, jnp.int32)
    lo_m = jnp.full((16,), 127, jnp.int32)
    g_m = jnp.full((16,), 7, jnp.int32)
    c3 = jnp.full((16,), 3, jnp.int32)
    c7 = jnp.full((16,), 7, jnp.int32)

    def dsti(i, _):
        for j in range(GSZ // 16):
            m = dst_v[i, pl.ds(j * 16, 16)]
            t = lanes + jnp.full((16,), i * GSZ + j * 16, jnp.int32)
            row = basev + lax.div(t, lvec)
            dst_v[i, pl.ds(j * 16, 16)] = jnp.where(m > zeros16, row, junkv)
            v = ids_v[i, pl.ds(j * 16, 16)]
            idx2 = ((v & hi_m)
                    | lax.shift_left(v & lo_m, c3)
                    | (lax.shift_right_logical(v, c7) & g_m))
            ids_v[i, pl.ds(j * 16, 16)] = idx2
        return ()
    lax.fori_loop(0, G, dsti, ())

    def gather(g, k):
        pltpu.make_async_copy(
            table_hbm.at[ids_v.at[g]], buf_v.at[k], sems[k]).start()

    def wait_scatter(g, k):
        pltpu.make_async_copy(
            table_hbm.at[ids_v.at[g]], buf_v.at[k], sems[k]).wait()
        pltpu.sync_copy(buf_v.at[k], acc_sh.at[dst_v.at[g]], add=True)

    for k in range(NBUF):
        gather(k, k)

    def ring(it, _):
        g0 = it * NBUF
        for k in range(NBUF):
            wait_scatter(g0 + k, k)
            gather(g0 + NBUF + k, k)
        return ()
    lax.fori_loop(0, G // NBUF - 1, ring, ())

    for k in range(NBUF):
        wait_scatter(G - NBUF + k, k)

    pltpu.sync_copy(acc_sh.at[pl.ds(base, RW)],
                    out_hbm.at[pl.ds(wid * RW, RW)])


TBLK = 8192
TGRID = -(-VOCAB // TBLK)
VPAD = TGRID * TBLK


def _tc_transpose_body(t_ref, o_ref):
    x = t_ref[...]
    for t in range(TBLK // 1024):
        xs = jnp.concatenate(
            [x[:, 1024 * t + 128 * g:1024 * t + 128 * (g + 1)]
             for g in range(8)], axis=0)
        o_ref[pl.ds(128 * t, 128), :] = xs.T


_tc_linearize = pl.pallas_call(
    _tc_transpose_body,
    grid=(TGRID,),
    in_specs=[pl.BlockSpec((D, TBLK), lambda k: (0, k))],
    out_specs=pl.BlockSpec((TBLK * D // 128, 128), lambda k: (k, 0)),
    out_shape=jax.ShapeDtypeStruct((VPAD * D // 128, 128), jnp.float32),
)


def kernel(input_ids, attention_mask, token_type_ids, embed_table):
    del token_type_ids
    ids = input_ids.astype(jnp.int32).reshape(B * L // GSZ, GSZ)
    mask = attention_mask.astype(jnp.int32).reshape(B * L // GSZ, GSZ)
    table_lin = lax.optimization_barrier(
        _tc_linearize(embed_table.T).reshape(VPAD * D)).reshape(VPAD, D)
    return _sc_pool(ids, mask, table_lin)


# speedup vs baseline: 7.5565x; 1.1334x over previous
import functools

import jax
import jax.numpy as jnp
from jax import lax
from jax.experimental import pallas as pl
from jax.experimental.pallas import tpu as pltpu
from jax.experimental.pallas import tpu_sc as plsc

VOCAB = 1000000
B = 4096
L = 200
D = 16
NC = 2
NS = 16
NW = NC * NS
RW = B // NW
TPW = RW * L
GSZ = 128
G = TPW // GSZ
NBUF = 8
ACCR = RW + 8

_mesh = plsc.VectorSubcoreMesh(core_axis_name="c", subcore_axis_name="s")


@functools.partial(
    pl.kernel,
    out_type=jax.ShapeDtypeStruct((B, D), jnp.float32),
    mesh=_mesh,
    scratch_types=[
        pltpu.VMEM((G, GSZ), jnp.int32),
        pltpu.VMEM((G, GSZ), jnp.int32),
        pltpu.VMEM((ACCR, D), jnp.float32),
        pltpu.VMEM((NBUF, GSZ, D), jnp.float32),
        pltpu.VMEM_SHARED((NS * ACCR, D), jnp.float32),
        [pltpu.SemaphoreType.DMA] * NBUF,
    ],
    compiler_params=pltpu.CompilerParams(use_tc_tiling_on_sc=False),
)
def _sc_pool(ids_hbm, mask_hbm, table_hbm, out_hbm,
             ids_v, dst_v, zero_v, buf_v, acc_sh, sems):
    c = lax.axis_index("c")
    s = lax.axis_index("s")
    wid = c * NS + s
    base = s * ACCR
    junk = base + RW

    pltpu.sync_copy(ids_hbm.at[pl.ds(wid * G, G)], ids_v)
    pltpu.sync_copy(mask_hbm.at[pl.ds(wid * G, G)], dst_v)

    def zrow(i, _):
        zero_v[i, :] = jnp.zeros((D,), jnp.float32)
        return ()
    lax.fori_loop(0, ACCR, zrow, ())
    pltpu.sync_copy(zero_v, acc_sh.at[pl.ds(base, ACCR)])

    lanes = lax.iota(jnp.int32, 16)
    zeros16 = jnp.zeros((16,), jnp.int32)
    lvec = jnp.full((16,), L, jnp.int32)
    basev = jnp.full((16,), base, jnp.int32)
    junkv = jnp.full((16,), junk, jnp.int32)
    hi_m = jnp.full((16,), -1024, jnp.int32)
    lo_m = jnp.full((16,), 127, jnp.int32)
    g_m = jnp.full((16,), 7, jnp.int32)
    c3 = jnp.full((16,), 3, jnp.int32)
    c7 = jnp.full((16,), 7, jnp.int32)

    def dsti(i, _):
        for j in range(GSZ // 16):
            m = dst_v[i, pl.ds(j * 16, 16)]
            t = lanes + jnp.full((16,), i * GSZ + j * 16, jnp.int32)
            row = basev + lax.div(t, lvec)
            dst_v[i, pl.ds(j * 16, 16)] = jnp.where(m > zeros16, row, junkv)
            v = ids_v[i, pl.ds(j * 16, 16)]
            idx2 = ((v & hi_m)
                    | lax.shift_left(v & lo_m, c3)
                    | (lax.shift_right_logical(v, c7) & g_m))
            ids_v[i, pl.ds(j * 16, 16)] = idx2
        return ()
    lax.fori_loop(0, G, dsti, ())

    def gather(g, k):
        pltpu.make_async_copy(
            table_hbm.at[ids_v.at[g]], buf_v.at[k], sems[k]).start()

    def wait_scatter(g, k):
        pltpu.make_async_copy(
            table_hbm.at[ids_v.at[g]], buf_v.at[k], sems[k]).wait()
        pltpu.sync_copy(buf_v.at[k], acc_sh.at[dst_v.at[g]], add=True)

    for k in range(NBUF):
        gather(k, k)

    def ring(it, _):
        g0 = it * NBUF
        for k in range(NBUF):
            wait_scatter(g0 + k, k)
            gather(g0 + NBUF + k, k)
        return ()
    lax.fori_loop(0, G // NBUF - 1, ring, ())

    for k in range(NBUF):
        wait_scatter(G - NBUF + k, k)

    pltpu.sync_copy(acc_sh.at[pl.ds(base, RW)],
                    out_hbm.at[pl.ds(wid * RW, RW)])


TBLK = 16384
TGRID = -(-VOCAB // TBLK)
VPAD = TGRID * TBLK


def _tc_transpose_body(t_ref, o_ref):
    x = t_ref[...]
    for t in range(TBLK // 1024):
        xs = jnp.concatenate(
            [x[:, 1024 * t + 128 * g:1024 * t + 128 * (g + 1)]
             for g in range(8)], axis=0)
        o_ref[pl.ds(128 * t, 128), :] = xs.T


_tc_linearize = pl.pallas_call(
    _tc_transpose_body,
    grid=(TGRID,),
    in_specs=[pl.BlockSpec((D, TBLK), lambda k: (0, k))],
    out_specs=pl.BlockSpec((TBLK * D // 128, 128), lambda k: (k, 0)),
    out_shape=jax.ShapeDtypeStruct((VPAD * D // 128, 128), jnp.float32),
)


def kernel(input_ids, attention_mask, token_type_ids, embed_table):
    del token_type_ids
    ids = input_ids.astype(jnp.int32).reshape(B * L // GSZ, GSZ)
    mask = attention_mask.astype(jnp.int32).reshape(B * L // GSZ, GSZ)
    table_lin = lax.optimization_barrier(
        _tc_linearize(embed_table.T).reshape(VPAD * D)).reshape(VPAD, D)
    return _sc_pool(ids, mask, table_lin)


# speedup vs baseline: 11.7294x; 1.5522x over previous
import functools

import jax
import jax.numpy as jnp
from jax import lax
from jax.experimental import pallas as pl
from jax.experimental.pallas import tpu as pltpu
from jax.experimental.pallas import tpu_sc as plsc

VOCAB = 1000000
B = 4096
L = 200
D = 16
NC = 2
NS = 16
NW = NC * NS
RW = B // NW
TPW = RW * L
GSZ = 128
G = TPW // GSZ
NBUF = 8
ACCR = RW + 8

_mesh = plsc.VectorSubcoreMesh(core_axis_name="c", subcore_axis_name="s")


@functools.partial(
    pl.kernel,
    out_type=jax.ShapeDtypeStruct((B, D), jnp.float32),
    mesh=_mesh,
    scratch_types=[
        pltpu.VMEM((G, GSZ), jnp.int32),
        pltpu.VMEM((G, GSZ), jnp.int32),
        pltpu.VMEM((ACCR, D), jnp.float32),
        pltpu.VMEM((NBUF, GSZ, D), jnp.float32),
        pltpu.VMEM_SHARED((NS * ACCR, D), jnp.float32),
        [pltpu.SemaphoreType.DMA] * NBUF,
    ],
    compiler_params=pltpu.CompilerParams(use_tc_tiling_on_sc=False),
)
def _sc_pool(ids_hbm, mask_hbm, table_hbm, out_hbm,
             ids_v, dst_v, zero_v, buf_v, acc_sh, sems):
    c = lax.axis_index("c")
    s = lax.axis_index("s")
    wid = c * NS + s
    base = s * ACCR
    junk = base + RW

    pltpu.sync_copy(ids_hbm.at[pl.ds(wid * G, G)], ids_v)
    pltpu.sync_copy(mask_hbm.at[pl.ds(wid * G, G)], dst_v)

    def zrow(i, _):
        zero_v[i, :] = jnp.zeros((D,), jnp.float32)
        return ()
    lax.fori_loop(0, ACCR, zrow, ())
    pltpu.sync_copy(zero_v, acc_sh.at[pl.ds(base, ACCR)])

    lanes = lax.iota(jnp.int32, 16)
    zeros16 = jnp.zeros((16,), jnp.int32)
    lvec = jnp.full((16,), L, jnp.int32)
    basev = jnp.full((16,), base, jnp.int32)
    junkv = jnp.full((16,), junk, jnp.int32)
    hi_m = jnp.full((16,), -1024, jnp.int32)
    lo_m = jnp.full((16,), 127, jnp.int32)
    g_m = jnp.full((16,), 7, jnp.int32)
    c3 = jnp.full((16,), 3, jnp.int32)
    c7 = jnp.full((16,), 7, jnp.int32)

    def dsti(i):
        for j in range(GSZ // 16):
            m = dst_v[i, pl.ds(j * 16, 16)]
            t = lanes + jnp.full((16,), i * GSZ + j * 16, jnp.int32)
            row = basev + lax.div(t, lvec)
            dst_v[i, pl.ds(j * 16, 16)] = jnp.where(
                m > zeros16, row, junkv + (t & c7))
            v = ids_v[i, pl.ds(j * 16, 16)]
            idx2 = ((v & hi_m)
                    | lax.shift_left(v & lo_m, c3)
                    | (lax.shift_right_logical(v, c7) & g_m))
            ids_v[i, pl.ds(j * 16, 16)] = idx2

    def gather(g, k):
        pltpu.make_async_copy(
            table_hbm.at[ids_v.at[g]], buf_v.at[k], sems[k]).start()

    def wait_scatter(g, k):
        pltpu.make_async_copy(
            table_hbm.at[ids_v.at[g]], buf_v.at[k], sems[k]).wait()
        pltpu.sync_copy(buf_v.at[k], acc_sh.at[dst_v.at[g]], add=True)

    for k in range(NBUF):
        dsti(k)
        gather(k, k)

    def ring(it, _):
        g0 = it * NBUF
        for k in range(NBUF):
            wait_scatter(g0 + k, k)
            dsti(g0 + NBUF + k)
            gather(g0 + NBUF + k, k)
        return ()
    lax.fori_loop(0, G // NBUF - 1, ring, ())

    for k in range(NBUF):
        wait_scatter(G - NBUF + k, k)

    pltpu.sync_copy(acc_sh.at[pl.ds(base, RW)],
                    out_hbm.at[pl.ds(wid * RW, RW)])


TBLK = 32768
TGRID = -(-VOCAB // TBLK)
VPAD = TGRID * TBLK


def _tc_transpose_body(t_ref, o_ref):
    x = t_ref[...]
    for t in range(TBLK // 1024):
        xs = jnp.concatenate(
            [x[:, 1024 * t + 128 * g:1024 * t + 128 * (g + 1)]
             for g in range(8)], axis=0)
        o_ref[pl.ds(128 * t, 128), :] = xs.T


_tc_linearize = pl.pallas_call(
    _tc_transpose_body,
    grid=(TGRID,),
    in_specs=[pl.BlockSpec((D, TBLK), lambda k: (0, k))],
    out_specs=pl.BlockSpec((TBLK * D // 128, 128), lambda k: (k, 0)),
    out_shape=jax.ShapeDtypeStruct((VPAD * D // 128, 128), jnp.float32),
)


def kernel(input_ids, attention_mask, token_type_ids, embed_table):
    del token_type_ids
    ids = input_ids.astype(jnp.int32).reshape(B * L // GSZ, GSZ)
    mask = attention_mask.astype(jnp.int32).reshape(B * L // GSZ, GSZ)
    table_lin = lax.optimization_barrier(
        _tc_linearize(embed_table.T).reshape(VPAD * D)).reshape(VPAD, D)
    return _sc_pool(ids, mask, table_lin)


# speedup vs baseline: 14.4489x; 1.2318x over previous
import functools

import jax
import jax.numpy as jnp
from jax import lax
from jax.experimental import pallas as pl
from jax.experimental.pallas import tpu as pltpu
from jax.experimental.pallas import tpu_sc as plsc

VOCAB = 1000000
B = 4096
L = 200
D = 16
NC = 2
NS = 16
NW = NC * NS
RW = B // NW
TPW = RW * L
GSZ = 128
G = TPW // GSZ
NBUF = 10
ACCR = RW + 8

_mesh = plsc.VectorSubcoreMesh(core_axis_name="c", subcore_axis_name="s")


@functools.partial(
    pl.kernel,
    out_type=jax.ShapeDtypeStruct((B, D), jnp.float32),
    mesh=_mesh,
    scratch_types=[
        pltpu.VMEM((G, GSZ), jnp.int32),
        pltpu.VMEM((G, GSZ), jnp.int32),
        pltpu.VMEM((ACCR, D), jnp.float32),
        pltpu.VMEM((NBUF, GSZ, D), jnp.float32),
        pltpu.VMEM_SHARED((NS * ACCR, D), jnp.float32),
        [pltpu.SemaphoreType.DMA] * NBUF,
    ],
    compiler_params=pltpu.CompilerParams(use_tc_tiling_on_sc=False),
)
def _sc_pool(ids_hbm, mask_hbm, table_hbm, out_hbm,
             ids_v, dst_v, zero_v, buf_v, acc_sh, sems):
    c = lax.axis_index("c")
    s = lax.axis_index("s")
    wid = c * NS + s
    base = s * ACCR
    junk = base + RW

    pltpu.sync_copy(ids_hbm.at[pl.ds(0, L), pl.ds(wid * RW, RW)], ids_v)
    pltpu.sync_copy(mask_hbm.at[pl.ds(0, L), pl.ds(wid * RW, RW)], dst_v)

    def zrow(i, _):
        zero_v[i, :] = jnp.zeros((D,), jnp.float32)
        return ()
    lax.fori_loop(0, ACCR, zrow, ())
    pltpu.sync_copy(zero_v, acc_sh.at[pl.ds(base, ACCR)])

    lanes = lax.iota(jnp.int32, 16)
    zeros16 = jnp.zeros((16,), jnp.int32)
    c127 = jnp.full((16,), 127, jnp.int32)
    basev = jnp.full((16,), base, jnp.int32)
    junkv = jnp.full((16,), junk, jnp.int32)
    hi_m = jnp.full((16,), -1024, jnp.int32)
    lo_m = jnp.full((16,), 127, jnp.int32)
    g_m = jnp.full((16,), 7, jnp.int32)
    c3 = jnp.full((16,), 3, jnp.int32)
    c7 = jnp.full((16,), 7, jnp.int32)

    def dsti(i):
        for j in range(GSZ // 16):
            m = dst_v[i, pl.ds(j * 16, 16)]
            t = lanes + jnp.full((16,), i * GSZ + j * 16, jnp.int32)
            row = basev + (t & c127)
            dst_v[i, pl.ds(j * 16, 16)] = jnp.where(
                m > zeros16, row, junkv + (t & c7))
            v = ids_v[i, pl.ds(j * 16, 16)]
            idx2 = ((v & hi_m)
                    | lax.shift_left(v & lo_m, c3)
                    | (lax.shift_right_logical(v, c7) & g_m))
            ids_v[i, pl.ds(j * 16, 16)] = idx2

    def gather(g, k):
        pltpu.make_async_copy(
            table_hbm.at[ids_v.at[g]], buf_v.at[k], sems[k]).start()

    def wait_scatter(g, k):
        pltpu.make_async_copy(
            table_hbm.at[ids_v.at[g]], buf_v.at[k], sems[k]).wait()
        pltpu.sync_copy(buf_v.at[k], acc_sh.at[dst_v.at[g]], add=True)

    for k in range(NBUF):
        dsti(k)
        gather(k, k)

    def ring(it, _):
        g0 = it * NBUF
        for k in range(NBUF):
            wait_scatter(g0 + k, k)
            dsti(g0 + NBUF + k)
            gather(g0 + NBUF + k, k)
        return ()
    lax.fori_loop(0, G // NBUF - 1, ring, ())

    for k in range(NBUF):
        wait_scatter(G - NBUF + k, k)

    pltpu.sync_copy(acc_sh.at[pl.ds(base, RW)],
                    out_hbm.at[pl.ds(wid * RW, RW)])


TBLK = 131072
TGRID = -(-VOCAB // TBLK)
VPAD = TGRID * TBLK


def _tc_transpose_body(t_ref, o_ref):
    x = t_ref[...]
    for t in range(TBLK // 1024):
        xs = jnp.concatenate(
            [x[:, 1024 * t + 128 * g:1024 * t + 128 * (g + 1)]
             for g in range(8)], axis=0)
        o_ref[pl.ds(128 * t, 128), :] = xs.T


_tc_linearize = pl.pallas_call(
    _tc_transpose_body,
    grid=(TGRID,),
    in_specs=[pl.BlockSpec((D, TBLK), lambda k: (0, k))],
    out_specs=pl.BlockSpec((TBLK * D // 128, 128), lambda k: (k, 0)),
    out_shape=jax.ShapeDtypeStruct((VPAD * D // 128, 128), jnp.float32),
)


def kernel(input_ids, attention_mask, token_type_ids, embed_table):
    del token_type_ids
    ids = lax.optimization_barrier(
        input_ids.T.astype(jnp.int32).reshape(B * L)).reshape(L, B)
    mask = lax.optimization_barrier(
        attention_mask.T.astype(jnp.int32).reshape(B * L)).reshape(L, B)
    table_lin = lax.optimization_barrier(
        _tc_linearize(embed_table.T).reshape(VPAD * D)).reshape(VPAD, D)
    return _sc_pool(ids, mask, table_lin)
